# baseline (speedup 1.0000x reference)
import dataclasses
import functools

import jax
import jax.numpy as jnp
from jax import lax
from jax.experimental import pallas as pl
from jax.experimental.pallas import tpu as pltpu
from jax.experimental.pallas import tpu_sc as plsc

_N = 10000
_E = 320000
_D = 128
_H = 64
_G = 200
_C = 6

_CHUNK = 128
_NCHUNK = _E // _CHUNK
_NCORE = 2
_NSUB = 16
_GP = 256
_FLAT = _GP * _H
_PROWS = 400

_mesh = plsc.VectorSubcoreMesh(core_axis_name="c", subcore_axis_name="s")
_SC_PARAMS = pltpu.CompilerParams(use_tc_tiling_on_sc=False)
_SC_PARAMS_NOLAYOUT = (
    dataclasses.replace(_SC_PARAMS, needs_layout_passes=False)
    if "needs_layout_passes" in pltpu.CompilerParams.__dataclass_fields__
    else _SC_PARAMS)


_NK = 80
_EPAD = _NK * _CHUNK * _NCORE * _NSUB
_NACC = 10080


@functools.partial(
    pl.kernel,
    out_type=jax.ShapeDtypeStruct((_NCORE, _N, _H), jnp.float32),
    mesh=_mesh,
    scratch_types=[
        pltpu.VMEM((_NK, _CHUNK), jnp.int32),
        pltpu.VMEM((_NK, _CHUNK), jnp.int32),
    ] + [pltpu.VMEM((_CHUNK, _H), jnp.float32)] * 4 + [
        pltpu.VMEM((160, _H), jnp.float32),
        pltpu.VMEM_SHARED((_NACC, _H), jnp.float32),
    ] + [pltpu.SemaphoreType.DMA] * 4,
    compiler_params=_SC_PARAMS,
)
def _agg_kernel(y_hbm, ei_hbm, out_hbm, src_v, dst_v, *rest):
    bufs = rest[:4]
    zbuf_v, acc_sp = rest[4:6]
    gsem = rest[6:10]
    c = lax.axis_index("c")
    s = lax.axis_index("s")
    t = s * _NCORE + c
    zero16 = jnp.zeros((16,), jnp.float32)

    @pl.loop(0, 160)
    def _(r):
        for j in range(_H // 16):
            zbuf_v[r, pl.ds(16 * j, 16)] = zero16

    pltpu.sync_copy(ei_hbm.at[0, t], src_v)
    pltpu.sync_copy(ei_hbm.at[1, t], dst_v)

    nzero = jnp.where(s == _NSUB - 1, 3, 4)

    def _zbody(i, carry):
        pltpu.sync_copy(zbuf_v, acc_sp.at[pl.ds(640 * s + 160 * i, 160)])
        return carry

    lax.fori_loop(0, nzero, _zbody, 0)

    plsc.subcore_barrier()

    for m in range(4):
        pltpu.async_copy(y_hbm.at[src_v.at[m]], bufs[m], gsem[m])

    def _quad(q, carry):
        for slot in range(4):
            k = 4 * q + slot
            m = slot
            pltpu.make_async_copy(
                y_hbm.at[src_v.at[k]], bufs[m], gsem[m]).wait()
            pltpu.sync_copy(bufs[m], acc_sp.at[dst_v.at[k]], add=True)

            @pl.when(k + 4 < _NK)
            def _():
                pltpu.async_copy(
                    y_hbm.at[src_v.at[k + 4]], bufs[m], gsem[m])

        return carry

    lax.fori_loop(0, _NK // 4, _quad, 0)
    plsc.subcore_barrier()

    @pl.when(s < _NSUB - 1)
    def _():
        pltpu.sync_copy(acc_sp.at[pl.ds(640 * s, 640)],
                        out_hbm.at[c, pl.ds(640 * s, 640)])

    @pl.when(s == _NSUB - 1)
    def _():
        pltpu.sync_copy(acc_sp.at[pl.ds(9600, 400)],
                        out_hbm.at[c, pl.ds(9600, 400)])


@functools.partial(
    pl.kernel,
    out_type=(
        jax.ShapeDtypeStruct((_NCORE * _NSUB, _FLAT), jnp.float32),
        jax.ShapeDtypeStruct((_NCORE * _NSUB, _FLAT), jnp.float32),
        jax.ShapeDtypeStruct((_NCORE * _NSUB, _FLAT), jnp.float32),
    ),
    mesh=_mesh,
    scratch_types=[
        pltpu.VMEM((_PROWS, _H), jnp.float32),
        pltpu.VMEM((_PROWS,), jnp.int32),
        pltpu.VMEM((_FLAT,), jnp.float32),
        pltpu.VMEM((_FLAT,), jnp.float32),
        pltpu.VMEM((_FLAT,), jnp.float32),
    ],
    compiler_params=_SC_PARAMS_NOLAYOUT,
)
def _pool_kernel(h_hbm, b_hbm, osum, omax, ocnt,
                 rows_v, bid_v, sum_v, max_v, cnt_v):
    c = lax.axis_index("c")
    s = lax.axis_index("s")
    wid = s * _NCORE + c
    zero16 = jnp.zeros((16,), jnp.float32)
    ninf16 = jnp.full((16,), -3.0e38, jnp.float32)

    @pl.loop(0, _FLAT // 16)
    def _(i):
        sum_v[pl.ds(16 * i, 16)] = zero16
        max_v[pl.ds(16 * i, 16)] = ninf16
        cnt_v[pl.ds(16 * i, 16)] = zero16

    @pl.when(wid < _N // _PROWS)
    def _():
        base = _PROWS * wid
        pltpu.sync_copy(h_hbm.at[pl.ds(base, _PROWS)], rows_v)
        pltpu.sync_copy(b_hbm.at[pl.ds(base, _PROWS)], bid_v)
        lane = lax.iota(jnp.int32, 16)
        ones16 = jnp.ones((16,), jnp.float32)

        def _rbody(r, carry):
            b = plsc.load_gather(bid_v, [jnp.full((16,), r, jnp.int32)])
            b64 = b * _H
            for j in range(_H // 16):
                idx = b64 + (16 * j) + lane
                chunk = rows_v[r, pl.ds(16 * j, 16)]
                plsc.addupdate_scatter(sum_v, [idx], chunk)
                plsc.addupdate_scatter(cnt_v, [idx], ones16)
                old = plsc.load_gather(max_v, [idx])
                plsc.store_scatter(max_v, [idx], jnp.maximum(old, chunk))
            return carry

        lax.fori_loop(0, _PROWS, _rbody, 0)

    pltpu.sync_copy(sum_v, osum.at[wid])
    pltpu.sync_copy(max_v, omax.at[wid])
    pltpu.sync_copy(cnt_v, ocnt.at[wid])


_PREC = lax.Precision.DEFAULT


def _proj_body(x_ref, w_ref, o_ref):
    o_ref[...] = jnp.dot(x_ref[...], w_ref[...],
                         preferred_element_type=jnp.float32, precision=_PREC)


def _mlp_body(has_proj, y_ref, p0_ref, p1_ref, eps_ref, ba_ref, wb_ref,
              bb_ref, g_ref, be_ref, rm_ref, rv_ref, *rest):
    if has_proj:
        wn_ref, o_ref = rest
    else:
        (o_ref,) = rest
    z = ((1.0 + eps_ref[0, 0]) * y_ref[...] + p0_ref[...] + p1_ref[...]
         + ba_ref[...])
    a = jnp.maximum(z, 0.0)
    u = jnp.dot(a, wb_ref[...], preferred_element_type=jnp.float32,
                precision=_PREC) + bb_ref[...]
    v = jnp.maximum(u, 0.0)
    hh = ((v - rm_ref[...]) / jnp.sqrt(rv_ref[...] + 1e-5) * g_ref[...]
          + be_ref[...])
    if has_proj:
        o_ref[...] = jnp.dot(hh, wn_ref[...],
                             preferred_element_type=jnp.float32,
                             precision=_PREC)
    else:
        o_ref[...] = hh


def _final_body(s_ref, m_ref, c_ref, wf_ref, bf_ref, o_ref):
    ssum = s_ref[0]
    mmax = m_ref[0]
    csum = c_ref[0]
    for i in range(1, _NCORE * _NSUB):
        ssum = ssum + s_ref[i]
        mmax = jnp.maximum(mmax, m_ref[i])
        csum = csum + c_ref[i]
    ssum = ssum[:_G]
    mmax = mmax[:_G]
    csum = csum[:_G]
    mean = ssum / jnp.maximum(csum, 1.0)
    mx = jnp.where(csum > 0.0, mmax, 0.0)
    pooled = jnp.concatenate([mean, mx], axis=1)
    logits = jnp.dot(pooled, wf_ref[...], preferred_element_type=jnp.float32,
                     precision=_PREC) + bf_ref[...]
    lmax = jnp.max(logits, axis=1, keepdims=True)
    shifted = logits - lmax
    lse = jnp.log(jnp.sum(jnp.exp(shifted), axis=1, keepdims=True))
    o_ref[...] = shifted - lse


def _proj(x, w):
    return pl.pallas_call(
        _proj_body,
        out_shape=jax.ShapeDtypeStruct((_N, _H), jnp.float32),
    )(x, w)


def _mlp(y, p0, p1, eps, ba, wb, bb, g, be, rm, rv, wn):
    args = [y, p0, p1, eps.reshape(1, 1), ba.reshape(1, _H), wb,
            bb.reshape(1, _H), g.reshape(1, _H), be.reshape(1, _H),
            rm.reshape(1, _H), rv.reshape(1, _H)]
    if wn is not None:
        args.append(wn)
    return pl.pallas_call(
        functools.partial(_mlp_body, wn is not None),
        out_shape=jax.ShapeDtypeStruct((_N, _H), jnp.float32),
    )(*args)


def _final(su, mx, ct, wf, bf):
    return pl.pallas_call(
        _final_body,
        out_shape=jax.ShapeDtypeStruct((_G, _C), jnp.float32),
    )(su, mx, ct, wf, bf.reshape(1, _C))


def kernel(x, edge_index, batch,
           W1a, b1a, W1b, b1b,
           W2a, b2a, W2b, b2b,
           W3a, b3a, W3b, b3b,
           eps1, eps2, eps3, eps4,
           g1, be1, g2, be2, g3, be3, g4, be4,
           rm1, rv1, rm2, rv2, rm3, rv3, rm4, rv4,
           Wf, bf):
    pad = jnp.concatenate(
        [jnp.zeros((1, _EPAD - _E), jnp.int32),
         jnp.full((1, _EPAD - _E), _N, jnp.int32)], axis=0)
    ei2 = jnp.concatenate([edge_index, pad], axis=1).reshape(
        2, _NCORE * _NSUB, _NK, _CHUNK)

    y1 = _proj(x, W1a)
    p = _agg_kernel(y1, ei2)
    y2 = _mlp(y1, p[0], p[1], eps1, b1a, W1b, b1b, g1, be1, rm1, rv1, W2a)
    p = _agg_kernel(y2, ei2)
    y3 = _mlp(y2, p[0], p[1], eps2, b2a, W2b, b2b, g2, be2, rm2, rv2, W3a)
    p = _agg_kernel(y3, ei2)
    y4 = _mlp(y3, p[0], p[1], eps3, b3a, W3b, b3b, g3, be3, rm3, rv3, W3a)
    p = _agg_kernel(y4, ei2)
    h4 = _mlp(y4, p[0], p[1], eps4, b3a, W3b, b3b, g4, be4, rm4, rv4, None)

    su, mx, ct = _pool_kernel(h4, batch)
    su = su.reshape(_NCORE * _NSUB, _GP, _H)
    mx = mx.reshape(_NCORE * _NSUB, _GP, _H)
    ct = ct.reshape(_NCORE * _NSUB, _GP, _H)
    return _final(su, mx, ct, Wf, bf)


# speedup vs baseline: 2.1771x; 2.1771x over previous
import dataclasses
import functools

import jax
import jax.numpy as jnp
from jax import lax
from jax.experimental import pallas as pl
from jax.experimental.pallas import tpu as pltpu
from jax.experimental.pallas import tpu_sc as plsc

_N = 10000
_E = 320000
_D = 128
_H = 64
_G = 200
_C = 6

_CHUNK = 128
_NCHUNK = _E // _CHUNK
_NCORE = 2
_NSUB = 16
_GP = 256
_FLAT = _GP * _H
_PROWS = 400

_mesh = plsc.VectorSubcoreMesh(core_axis_name="c", subcore_axis_name="s")
_SC_PARAMS = pltpu.CompilerParams(use_tc_tiling_on_sc=False)
_SC_PARAMS_NOLAYOUT = (
    dataclasses.replace(_SC_PARAMS, needs_layout_passes=False)
    if "needs_layout_passes" in pltpu.CompilerParams.__dataclass_fields__
    else _SC_PARAMS)


_NK = 80
_EPAD = _NK * _CHUNK * _NCORE * _NSUB
_NACC = 10080


@functools.partial(
    pl.kernel,
    out_type=jax.ShapeDtypeStruct((_NCORE, _N, _H), jnp.float32),
    mesh=_mesh,
    scratch_types=[
        pltpu.VMEM((2 * _NK, _CHUNK), jnp.int32),
    ] + [pltpu.VMEM((_CHUNK, _H), jnp.float32)] * 3 + [
        pltpu.VMEM_SHARED((_N, _H), jnp.float32),
        pltpu.VMEM_SHARED((_NACC, _H), jnp.float32),
    ] + [pltpu.SemaphoreType.DMA] * 3,
    compiler_params=_SC_PARAMS,
)
def _agg_kernel(y_hbm, ei_hbm, out_hbm, idx_v, *rest):
    bufs = rest[:3]
    ytab_sp, acc_sp = rest[3:5]
    gsem = rest[5:8]
    c = lax.axis_index("c")
    s = lax.axis_index("s")
    t = s * _NCORE + c
    zero16 = jnp.zeros((16,), jnp.float32)

    @pl.loop(0, _CHUNK)
    def _(r):
        for j in range(_H // 16):
            bufs[0][r, pl.ds(16 * j, 16)] = zero16

    pltpu.sync_copy(ei_hbm.at[0, t], idx_v.at[pl.ds(0, _NK)])
    pltpu.sync_copy(ei_hbm.at[1, t], idx_v.at[pl.ds(_NK, _NK)])

    @pl.when(s < _NSUB - 1)
    def _():
        for i in range(5):
            pltpu.sync_copy(bufs[0],
                            acc_sp.at[pl.ds(640 * s + 128 * i, 128)])
        pltpu.sync_copy(y_hbm.at[pl.ds(640 * s, 640)],
                        ytab_sp.at[pl.ds(640 * s, 640)])

    @pl.when(s == _NSUB - 1)
    def _():
        for i in range(3):
            pltpu.sync_copy(bufs[0],
                            acc_sp.at[pl.ds(9600 + 128 * i, 128)])
        pltpu.sync_copy(bufs[0].at[pl.ds(0, 96)],
                        acc_sp.at[pl.ds(9984, 96)])
        pltpu.sync_copy(y_hbm.at[pl.ds(9600, 400)],
                        ytab_sp.at[pl.ds(9600, 400)])

    plsc.subcore_barrier()

    for m in range(3):
        pltpu.async_copy(ytab_sp.at[idx_v.at[m]], bufs[m], gsem[m])

    def _tri(q, carry):
        for slot in range(3):
            k = 3 * q + slot
            m = slot
            pltpu.make_async_copy(
                ytab_sp.at[idx_v.at[k]], bufs[m], gsem[m]).wait()
            pltpu.sync_copy(bufs[m], acc_sp.at[idx_v.at[_NK + k]], add=True)

            @pl.when(k + 3 < _NK)
            def _():
                pltpu.async_copy(
                    ytab_sp.at[idx_v.at[k + 3]], bufs[m], gsem[m])

        return carry

    lax.fori_loop(0, _NK // 3, _tri, 0)
    for k in (78, 79):
        m = k % 3
        pltpu.make_async_copy(
            ytab_sp.at[idx_v.at[k]], bufs[m], gsem[m]).wait()
        pltpu.sync_copy(bufs[m], acc_sp.at[idx_v.at[_NK + k]], add=True)
    plsc.subcore_barrier()

    @pl.when(s < _NSUB - 1)
    def _():
        pltpu.sync_copy(acc_sp.at[pl.ds(640 * s, 640)],
                        out_hbm.at[c, pl.ds(640 * s, 640)])

    @pl.when(s == _NSUB - 1)
    def _():
        pltpu.sync_copy(acc_sp.at[pl.ds(9600, 400)],
                        out_hbm.at[c, pl.ds(9600, 400)])


@functools.partial(
    pl.kernel,
    out_type=(
        jax.ShapeDtypeStruct((_NCORE * _NSUB, _FLAT), jnp.float32),
        jax.ShapeDtypeStruct((_NCORE * _NSUB, _FLAT), jnp.float32),
        jax.ShapeDtypeStruct((_NCORE * _NSUB, _FLAT), jnp.float32),
    ),
    mesh=_mesh,
    scratch_types=[
        pltpu.VMEM((_PROWS, _H), jnp.float32),
        pltpu.VMEM((_PROWS,), jnp.int32),
        pltpu.VMEM((_FLAT,), jnp.float32),
        pltpu.VMEM((_FLAT,), jnp.float32),
        pltpu.VMEM((_FLAT,), jnp.float32),
    ],
    compiler_params=_SC_PARAMS_NOLAYOUT,
)
def _pool_kernel(h_hbm, b_hbm, osum, omax, ocnt,
                 rows_v, bid_v, sum_v, max_v, cnt_v):
    c = lax.axis_index("c")
    s = lax.axis_index("s")
    wid = s * _NCORE + c
    zero16 = jnp.zeros((16,), jnp.float32)
    ninf16 = jnp.full((16,), -3.0e38, jnp.float32)

    @pl.loop(0, _FLAT // 16)
    def _(i):
        sum_v[pl.ds(16 * i, 16)] = zero16
        max_v[pl.ds(16 * i, 16)] = ninf16
        cnt_v[pl.ds(16 * i, 16)] = zero16

    @pl.when(wid < _N // _PROWS)
    def _():
        base = _PROWS * wid
        pltpu.sync_copy(h_hbm.at[pl.ds(base, _PROWS)], rows_v)
        pltpu.sync_copy(b_hbm.at[pl.ds(base, _PROWS)], bid_v)
        lane = lax.iota(jnp.int32, 16)
        ones16 = jnp.ones((16,), jnp.float32)

        def _rbody(r, carry):
            b = plsc.load_gather(bid_v, [jnp.full((16,), r, jnp.int32)])
            b64 = b * _H
            for j in range(_H // 16):
                idx = b64 + (16 * j) + lane
                chunk = rows_v[r, pl.ds(16 * j, 16)]
                plsc.addupdate_scatter(sum_v, [idx], chunk)
                plsc.addupdate_scatter(cnt_v, [idx], ones16)
                old = plsc.load_gather(max_v, [idx])
                plsc.store_scatter(max_v, [idx], jnp.maximum(old, chunk))
            return carry

        lax.fori_loop(0, _PROWS, _rbody, 0)

    pltpu.sync_copy(sum_v, osum.at[wid])
    pltpu.sync_copy(max_v, omax.at[wid])
    pltpu.sync_copy(cnt_v, ocnt.at[wid])


_PREC = lax.Precision.DEFAULT


def _proj_body(x_ref, w_ref, o_ref):
    o_ref[...] = jnp.dot(x_ref[...], w_ref[...],
                         preferred_element_type=jnp.float32, precision=_PREC)


def _mlp_body(has_proj, y_ref, p0_ref, p1_ref, eps_ref, ba_ref, wb_ref,
              bb_ref, g_ref, be_ref, rm_ref, rv_ref, *rest):
    if has_proj:
        wn_ref, o_ref = rest
    else:
        (o_ref,) = rest
    z = ((1.0 + eps_ref[0, 0]) * y_ref[...] + p0_ref[...] + p1_ref[...]
         + ba_ref[...])
    a = jnp.maximum(z, 0.0)
    u = jnp.dot(a, wb_ref[...], preferred_element_type=jnp.float32,
                precision=_PREC) + bb_ref[...]
    v = jnp.maximum(u, 0.0)
    hh = ((v - rm_ref[...]) / jnp.sqrt(rv_ref[...] + 1e-5) * g_ref[...]
          + be_ref[...])
    if has_proj:
        o_ref[...] = jnp.dot(hh, wn_ref[...],
                             preferred_element_type=jnp.float32,
                             precision=_PREC)
    else:
        o_ref[...] = hh


def _final_body(s_ref, m_ref, c_ref, wf_ref, bf_ref, o_ref):
    ssum = s_ref[0]
    mmax = m_ref[0]
    csum = c_ref[0]
    for i in range(1, _NCORE * _NSUB):
        ssum = ssum + s_ref[i]
        mmax = jnp.maximum(mmax, m_ref[i])
        csum = csum + c_ref[i]
    ssum = ssum[:_G]
    mmax = mmax[:_G]
    csum = csum[:_G]
    mean = ssum / jnp.maximum(csum, 1.0)
    mx = jnp.where(csum > 0.0, mmax, 0.0)
    pooled = jnp.concatenate([mean, mx], axis=1)
    logits = jnp.dot(pooled, wf_ref[...], preferred_element_type=jnp.float32,
                     precision=_PREC) + bf_ref[...]
    lmax = jnp.max(logits, axis=1, keepdims=True)
    shifted = logits - lmax
    lse = jnp.log(jnp.sum(jnp.exp(shifted), axis=1, keepdims=True))
    o_ref[...] = shifted - lse


def _proj(x, w):
    return pl.pallas_call(
        _proj_body,
        out_shape=jax.ShapeDtypeStruct((_N, _H), jnp.float32),
    )(x, w)


def _mlp(y, p0, p1, eps, ba, wb, bb, g, be, rm, rv, wn):
    args = [y, p0, p1, eps.reshape(1, 1), ba.reshape(1, _H), wb,
            bb.reshape(1, _H), g.reshape(1, _H), be.reshape(1, _H),
            rm.reshape(1, _H), rv.reshape(1, _H)]
    if wn is not None:
        args.append(wn)
    return pl.pallas_call(
        functools.partial(_mlp_body, wn is not None),
        out_shape=jax.ShapeDtypeStruct((_N, _H), jnp.float32),
    )(*args)


def _final(su, mx, ct, wf, bf):
    return pl.pallas_call(
        _final_body,
        out_shape=jax.ShapeDtypeStruct((_G, _C), jnp.float32),
    )(su, mx, ct, wf, bf.reshape(1, _C))


def kernel(x, edge_index, batch,
           W1a, b1a, W1b, b1b,
           W2a, b2a, W2b, b2b,
           W3a, b3a, W3b, b3b,
           eps1, eps2, eps3, eps4,
           g1, be1, g2, be2, g3, be3, g4, be4,
           rm1, rv1, rm2, rv2, rm3, rv3, rm4, rv4,
           Wf, bf):
    pad = jnp.concatenate(
        [jnp.zeros((1, _EPAD - _E), jnp.int32),
         jnp.full((1, _EPAD - _E), _N, jnp.int32)], axis=0)
    ei2 = jnp.concatenate([edge_index, pad], axis=1).reshape(
        2, _NCORE * _NSUB, _NK, _CHUNK)

    y1 = _proj(x, W1a)
    p = _agg_kernel(y1, ei2)
    y2 = _mlp(y1, p[0], p[1], eps1, b1a, W1b, b1b, g1, be1, rm1, rv1, W2a)
    p = _agg_kernel(y2, ei2)
    y3 = _mlp(y2, p[0], p[1], eps2, b2a, W2b, b2b, g2, be2, rm2, rv2, W3a)
    p = _agg_kernel(y3, ei2)
    y4 = _mlp(y3, p[0], p[1], eps3, b3a, W3b, b3b, g3, be3, rm3, rv3, W3a)
    p = _agg_kernel(y4, ei2)
    h4 = _mlp(y4, p[0], p[1], eps4, b3a, W3b, b3b, g4, be4, rm4, rv4, None)

    su, mx, ct = _pool_kernel(h4, batch)
    su = su.reshape(_NCORE * _NSUB, _GP, _H)
    mx = mx.reshape(_NCORE * _NSUB, _GP, _H)
    ct = ct.reshape(_NCORE * _NSUB, _GP, _H)
    return _final(su, mx, ct, Wf, bf)


# speedup vs baseline: 2.3740x; 1.0905x over previous
import dataclasses
import functools

import jax
import jax.numpy as jnp
from jax import lax
from jax.experimental import pallas as pl
from jax.experimental.pallas import tpu as pltpu
from jax.experimental.pallas import tpu_sc as plsc

_N = 10000
_E = 320000
_D = 128
_H = 64
_G = 200
_C = 6

_CHUNK = 128
_NCHUNK = _E // _CHUNK
_NCORE = 2
_NSUB = 16
_GP = 256
_FLAT = _GP * _H
_PROWS = 400

_mesh = plsc.VectorSubcoreMesh(core_axis_name="c", subcore_axis_name="s")
_SC_PARAMS = pltpu.CompilerParams(use_tc_tiling_on_sc=False)
_SC_PARAMS_NOLAYOUT = (
    dataclasses.replace(_SC_PARAMS, needs_layout_passes=False)
    if "needs_layout_passes" in pltpu.CompilerParams.__dataclass_fields__
    else _SC_PARAMS)


_NK = 80
_EPAD = _NK * _CHUNK * _NCORE * _NSUB
_NACC = 10080


@functools.partial(
    pl.kernel,
    out_type=jax.ShapeDtypeStruct((_NCORE, _N, _H), jnp.float32),
    mesh=_mesh,
    scratch_types=[
        pltpu.VMEM((2 * _NK, _CHUNK), jnp.int32),
    ] + [pltpu.VMEM((_CHUNK, _H), jnp.float32)] * 3 + [
        pltpu.VMEM_SHARED((_N, _H), jnp.float32),
        pltpu.VMEM_SHARED((_NACC, _H), jnp.float32),
    ] + [pltpu.SemaphoreType.DMA] * 6,
    compiler_params=_SC_PARAMS,
)
def _agg_kernel(y_hbm, ei_hbm, out_hbm, idx_v, *rest):
    bufs = rest[:3]
    ytab_sp, acc_sp = rest[3:5]
    gsem = rest[5:8]
    ssem = rest[8:11]
    c = lax.axis_index("c")
    s = lax.axis_index("s")
    t = s * _NCORE + c
    zero16 = jnp.zeros((16,), jnp.float32)

    @pl.loop(0, _CHUNK)
    def _(r):
        for j in range(_H // 16):
            bufs[0][r, pl.ds(16 * j, 16)] = zero16

    pltpu.sync_copy(ei_hbm.at[0, t], idx_v.at[pl.ds(0, _NK)])
    pltpu.sync_copy(ei_hbm.at[1, t], idx_v.at[pl.ds(_NK, _NK)])

    @pl.when(s < _NSUB - 1)
    def _():
        for i in range(5):
            pltpu.sync_copy(bufs[0],
                            acc_sp.at[pl.ds(640 * s + 128 * i, 128)])
        pltpu.sync_copy(y_hbm.at[pl.ds(640 * s, 640)],
                        ytab_sp.at[pl.ds(640 * s, 640)])

    @pl.when(s == _NSUB - 1)
    def _():
        for i in range(3):
            pltpu.sync_copy(bufs[0],
                            acc_sp.at[pl.ds(9600 + 128 * i, 128)])
        pltpu.sync_copy(bufs[0].at[pl.ds(0, 96)],
                        acc_sp.at[pl.ds(9984, 96)])
        pltpu.sync_copy(y_hbm.at[pl.ds(9600, 400)],
                        ytab_sp.at[pl.ds(9600, 400)])

    plsc.subcore_barrier()

    for m in range(2):
        pltpu.async_copy(ytab_sp.at[idx_v.at[m]], bufs[m], gsem[m])

    def _tri(q, carry):
        for slot in range(3):
            k = 3 * q + slot
            m = slot
            m2 = (slot + 2) % 3
            pltpu.make_async_copy(
                ytab_sp.at[idx_v.at[k]], bufs[m], gsem[m]).wait()
            pltpu.async_copy(bufs[m], acc_sp.at[idx_v.at[_NK + k]], ssem[m],
                             add=True)

            @pl.when(k >= 1)
            def _():
                pltpu.make_async_copy(
                    bufs[m2], acc_sp.at[idx_v.at[_NK]], ssem[m2]).wait()

            @pl.when(k + 2 < _NK)
            def _():
                pltpu.async_copy(
                    ytab_sp.at[idx_v.at[k + 2]], bufs[m2], gsem[m2])

        return carry

    lax.fori_loop(0, _NK // 3, _tri, 0)
    for k in (78, 79):
        m = k % 3
        m2 = (k + 2) % 3
        pltpu.make_async_copy(
            ytab_sp.at[idx_v.at[k]], bufs[m], gsem[m]).wait()
        pltpu.async_copy(bufs[m], acc_sp.at[idx_v.at[_NK + k]], ssem[m],
                         add=True)
        pltpu.make_async_copy(
            bufs[m2], acc_sp.at[idx_v.at[_NK]], ssem[m2]).wait()
    pltpu.make_async_copy(bufs[1], acc_sp.at[idx_v.at[_NK]],
                          ssem[1]).wait()
    plsc.subcore_barrier()

    @pl.when(s < _NSUB - 1)
    def _():
        pltpu.sync_copy(acc_sp.at[pl.ds(640 * s, 640)],
                        out_hbm.at[c, pl.ds(640 * s, 640)])

    @pl.when(s == _NSUB - 1)
    def _():
        pltpu.sync_copy(acc_sp.at[pl.ds(9600, 400)],
                        out_hbm.at[c, pl.ds(9600, 400)])


@functools.partial(
    pl.kernel,
    out_type=(
        jax.ShapeDtypeStruct((_NCORE * _NSUB, _FLAT), jnp.float32),
        jax.ShapeDtypeStruct((_NCORE * _NSUB, _FLAT), jnp.float32),
        jax.ShapeDtypeStruct((_NCORE * _NSUB, _FLAT), jnp.float32),
    ),
    mesh=_mesh,
    scratch_types=[
        pltpu.VMEM((_PROWS, _H), jnp.float32),
        pltpu.VMEM((_PROWS,), jnp.int32),
        pltpu.VMEM((_FLAT,), jnp.float32),
        pltpu.VMEM((_FLAT,), jnp.float32),
        pltpu.VMEM((_FLAT,), jnp.float32),
    ],
    compiler_params=_SC_PARAMS_NOLAYOUT,
)
def _pool_kernel(h_hbm, b_hbm, osum, omax, ocnt,
                 rows_v, bid_v, sum_v, max_v, cnt_v):
    c = lax.axis_index("c")
    s = lax.axis_index("s")
    wid = s * _NCORE + c
    zero16 = jnp.zeros((16,), jnp.float32)
    ninf16 = jnp.full((16,), -3.0e38, jnp.float32)

    @pl.loop(0, _FLAT // 16)
    def _(i):
        sum_v[pl.ds(16 * i, 16)] = zero16
        max_v[pl.ds(16 * i, 16)] = ninf16
        cnt_v[pl.ds(16 * i, 16)] = zero16

    @pl.when(wid < _N // _PROWS)
    def _():
        base = _PROWS * wid
        pltpu.sync_copy(h_hbm.at[pl.ds(base, _PROWS)], rows_v)
        pltpu.sync_copy(b_hbm.at[pl.ds(base, _PROWS)], bid_v)
        lane = lax.iota(jnp.int32, 16)
        ones16 = jnp.ones((16,), jnp.float32)

        def _rbody(r, carry):
            b = plsc.load_gather(bid_v, [jnp.full((16,), r, jnp.int32)])
            b64 = b * _H
            for j in range(_H // 16):
                idx = b64 + (16 * j) + lane
                chunk = rows_v[r, pl.ds(16 * j, 16)]
                plsc.addupdate_scatter(sum_v, [idx], chunk)
                plsc.addupdate_scatter(cnt_v, [idx], ones16)
                old = plsc.load_gather(max_v, [idx])
                plsc.store_scatter(max_v, [idx], jnp.maximum(old, chunk))
            return carry

        lax.fori_loop(0, _PROWS, _rbody, 0)

    pltpu.sync_copy(sum_v, osum.at[wid])
    pltpu.sync_copy(max_v, omax.at[wid])
    pltpu.sync_copy(cnt_v, ocnt.at[wid])


_PREC = lax.Precision.DEFAULT


def _proj_body(x_ref, w_ref, o_ref):
    o_ref[...] = jnp.dot(x_ref[...], w_ref[...],
                         preferred_element_type=jnp.float32, precision=_PREC)


def _mlp_body(has_proj, y_ref, p0_ref, p1_ref, eps_ref, ba_ref, wb_ref,
              bb_ref, g_ref, be_ref, rm_ref, rv_ref, *rest):
    if has_proj:
        wn_ref, o_ref = rest
    else:
        (o_ref,) = rest
    z = ((1.0 + eps_ref[0, 0]) * y_ref[...] + p0_ref[...] + p1_ref[...]
         + ba_ref[...])
    a = jnp.maximum(z, 0.0)
    u = jnp.dot(a, wb_ref[...], preferred_element_type=jnp.float32,
                precision=_PREC) + bb_ref[...]
    v = jnp.maximum(u, 0.0)
    hh = ((v - rm_ref[...]) / jnp.sqrt(rv_ref[...] + 1e-5) * g_ref[...]
          + be_ref[...])
    if has_proj:
        o_ref[...] = jnp.dot(hh, wn_ref[...],
                             preferred_element_type=jnp.float32,
                             precision=_PREC)
    else:
        o_ref[...] = hh


def _final_body(s_ref, m_ref, c_ref, wf_ref, bf_ref, o_ref):
    ssum = s_ref[0]
    mmax = m_ref[0]
    csum = c_ref[0]
    for i in range(1, _NCORE * _NSUB):
        ssum = ssum + s_ref[i]
        mmax = jnp.maximum(mmax, m_ref[i])
        csum = csum + c_ref[i]
    ssum = ssum[:_G]
    mmax = mmax[:_G]
    csum = csum[:_G]
    mean = ssum / jnp.maximum(csum, 1.0)
    mx = jnp.where(csum > 0.0, mmax, 0.0)
    pooled = jnp.concatenate([mean, mx], axis=1)
    logits = jnp.dot(pooled, wf_ref[...], preferred_element_type=jnp.float32,
                     precision=_PREC) + bf_ref[...]
    lmax = jnp.max(logits, axis=1, keepdims=True)
    shifted = logits - lmax
    lse = jnp.log(jnp.sum(jnp.exp(shifted), axis=1, keepdims=True))
    o_ref[...] = shifted - lse


def _proj(x, w):
    return pl.pallas_call(
        _proj_body,
        out_shape=jax.ShapeDtypeStruct((_N, _H), jnp.float32),
    )(x, w)


def _mlp(y, p0, p1, eps, ba, wb, bb, g, be, rm, rv, wn):
    args = [y, p0, p1, eps.reshape(1, 1), ba.reshape(1, _H), wb,
            bb.reshape(1, _H), g.reshape(1, _H), be.reshape(1, _H),
            rm.reshape(1, _H), rv.reshape(1, _H)]
    if wn is not None:
        args.append(wn)
    return pl.pallas_call(
        functools.partial(_mlp_body, wn is not None),
        out_shape=jax.ShapeDtypeStruct((_N, _H), jnp.float32),
    )(*args)


def _final(su, mx, ct, wf, bf):
    return pl.pallas_call(
        _final_body,
        out_shape=jax.ShapeDtypeStruct((_G, _C), jnp.float32),
    )(su, mx, ct, wf, bf.reshape(1, _C))


def kernel(x, edge_index, batch,
           W1a, b1a, W1b, b1b,
           W2a, b2a, W2b, b2b,
           W3a, b3a, W3b, b3b,
           eps1, eps2, eps3, eps4,
           g1, be1, g2, be2, g3, be3, g4, be4,
           rm1, rv1, rm2, rv2, rm3, rv3, rm4, rv4,
           Wf, bf):
    pad = jnp.concatenate(
        [jnp.zeros((1, _EPAD - _E), jnp.int32),
         jnp.full((1, _EPAD - _E), _N, jnp.int32)], axis=0)
    ei2 = jnp.concatenate([edge_index, pad], axis=1).reshape(
        2, _NCORE * _NSUB, _NK, _CHUNK)

    y1 = _proj(x, W1a)
    p = _agg_kernel(y1, ei2)
    y2 = _mlp(y1, p[0], p[1], eps1, b1a, W1b, b1b, g1, be1, rm1, rv1, W2a)
    p = _agg_kernel(y2, ei2)
    y3 = _mlp(y2, p[0], p[1], eps2, b2a, W2b, b2b, g2, be2, rm2, rv2, W3a)
    p = _agg_kernel(y3, ei2)
    y4 = _mlp(y3, p[0], p[1], eps3, b3a, W3b, b3b, g3, be3, rm3, rv3, W3a)
    p = _agg_kernel(y4, ei2)
    h4 = _mlp(y4, p[0], p[1], eps4, b3a, W3b, b3b, g4, be4, rm4, rv4, None)

    su, mx, ct = _pool_kernel(h4, batch)
    su = su.reshape(_NCORE * _NSUB, _GP, _H)
    mx = mx.reshape(_NCORE * _NSUB, _GP, _H)
    ct = ct.reshape(_NCORE * _NSUB, _GP, _H)
    return _final(su, mx, ct, Wf, bf)


# speedup vs baseline: 2.3759x; 1.0008x over previous
import dataclasses
import functools

import jax
import jax.numpy as jnp
from jax import lax
from jax.experimental import pallas as pl
from jax.experimental.pallas import tpu as pltpu
from jax.experimental.pallas import tpu_sc as plsc

_N = 10000
_E = 320000
_D = 128
_H = 64
_G = 200
_C = 6

_CHUNK = 128
_NCORE = 2
_NSUB = 16
_GP = 256
_FLAT = _GP * _H
_PROWS = 400

_mesh = plsc.VectorSubcoreMesh(core_axis_name="c", subcore_axis_name="s")
_SC_PARAMS = pltpu.CompilerParams(use_tc_tiling_on_sc=False)
_SC_PARAMS_NOLAYOUT = (
    dataclasses.replace(_SC_PARAMS, needs_layout_passes=False)
    if "needs_layout_passes" in pltpu.CompilerParams.__dataclass_fields__
    else _SC_PARAMS)


_NK = 80
_EPAD = _NK * _CHUNK * _NCORE * _NSUB
_NACC = 10080


@functools.partial(
    pl.kernel,
    out_type=jax.ShapeDtypeStruct((_NCORE, _N, _H), jnp.float32),
    mesh=_mesh,
    scratch_types=[
        pltpu.VMEM((2 * _NK, _CHUNK), jnp.int32),
    ] + [pltpu.VMEM((_CHUNK, _H), jnp.float32)] * 3 + [
        pltpu.VMEM_SHARED((_N, _H), jnp.float32),
        pltpu.VMEM_SHARED((_NACC, _H), jnp.float32),
    ] + [pltpu.SemaphoreType.DMA] * 6,
    compiler_params=_SC_PARAMS,
)
def _agg_kernel(y_hbm, ei_hbm, out_hbm, idx_v, *rest):
    bufs = rest[:3]
    ytab_sp, acc_sp = rest[3:5]
    gsem = rest[5:8]
    ssem = rest[8:11]
    c = lax.axis_index("c")
    s = lax.axis_index("s")
    t = s * _NCORE + c
    zero16 = jnp.zeros((16,), jnp.float32)

    @pl.loop(0, _CHUNK)
    def _(r):
        for j in range(_H // 16):
            bufs[0][r, pl.ds(16 * j, 16)] = zero16

    pltpu.sync_copy(ei_hbm.at[0, t], idx_v.at[pl.ds(0, _NK)])
    pltpu.sync_copy(ei_hbm.at[1, t], idx_v.at[pl.ds(_NK, _NK)])

    @pl.when(s < _NSUB - 1)
    def _():
        for i in range(5):
            pltpu.sync_copy(bufs[0],
                            acc_sp.at[pl.ds(640 * s + 128 * i, 128)])
        pltpu.sync_copy(y_hbm.at[pl.ds(640 * s, 640)],
                        ytab_sp.at[pl.ds(640 * s, 640)])

    @pl.when(s == _NSUB - 1)
    def _():
        for i in range(3):
            pltpu.sync_copy(bufs[0],
                            acc_sp.at[pl.ds(9600 + 128 * i, 128)])
        pltpu.sync_copy(bufs[0].at[pl.ds(0, 96)],
                        acc_sp.at[pl.ds(9984, 96)])
        pltpu.sync_copy(y_hbm.at[pl.ds(9600, 400)],
                        ytab_sp.at[pl.ds(9600, 400)])

    plsc.subcore_barrier()

    for m in range(2):
        pltpu.async_copy(ytab_sp.at[idx_v.at[m]], bufs[m], gsem[m])

    def _tri(q, carry):
        for slot in range(3):
            k = 3 * q + slot
            m = slot
            m2 = (slot + 2) % 3
            pltpu.make_async_copy(
                ytab_sp.at[idx_v.at[k]], bufs[m], gsem[m]).wait()
            pltpu.async_copy(bufs[m], acc_sp.at[idx_v.at[_NK + k]], ssem[m],
                             add=True)

            @pl.when(k >= 1)
            def _():
                pltpu.make_async_copy(
                    bufs[m2], acc_sp.at[idx_v.at[_NK]], ssem[m2]).wait()

            @pl.when(k + 2 < _NK)
            def _():
                pltpu.async_copy(
                    ytab_sp.at[idx_v.at[k + 2]], bufs[m2], gsem[m2])

        return carry

    lax.fori_loop(0, _NK // 3, _tri, 0)
    for k in (78, 79):
        m = k % 3
        m2 = (k + 2) % 3
        pltpu.make_async_copy(
            ytab_sp.at[idx_v.at[k]], bufs[m], gsem[m]).wait()
        pltpu.async_copy(bufs[m], acc_sp.at[idx_v.at[_NK + k]], ssem[m],
                         add=True)
        pltpu.make_async_copy(
            bufs[m2], acc_sp.at[idx_v.at[_NK]], ssem[m2]).wait()
    pltpu.make_async_copy(bufs[1], acc_sp.at[idx_v.at[_NK]],
                          ssem[1]).wait()
    plsc.subcore_barrier()

    @pl.when(s < _NSUB - 1)
    def _():
        pltpu.sync_copy(acc_sp.at[pl.ds(640 * s, 640)],
                        out_hbm.at[c, pl.ds(640 * s, 640)])

    @pl.when(s == _NSUB - 1)
    def _():
        pltpu.sync_copy(acc_sp.at[pl.ds(9600, 400)],
                        out_hbm.at[c, pl.ds(9600, 400)])


@functools.partial(
    pl.kernel,
    out_type=(
        jax.ShapeDtypeStruct((_NCORE * _NSUB, _FLAT), jnp.float32),
        jax.ShapeDtypeStruct((_NCORE * _NSUB, _FLAT), jnp.float32),
        jax.ShapeDtypeStruct((_NCORE * _NSUB, _FLAT), jnp.float32),
    ),
    mesh=_mesh,
    scratch_types=[
        pltpu.VMEM((_PROWS, _H), jnp.float32),
        pltpu.VMEM((_PROWS,), jnp.int32),
        pltpu.VMEM((_FLAT,), jnp.float32),
        pltpu.VMEM((_FLAT,), jnp.float32),
        pltpu.VMEM((_FLAT,), jnp.float32),
    ],
    compiler_params=_SC_PARAMS_NOLAYOUT,
)
def _pool_kernel(h_hbm, b_hbm, osum, omax, ocnt,
                 rows_v, bid_v, sum_v, max_v, cnt_v):
    c = lax.axis_index("c")
    s = lax.axis_index("s")
    wid = s * _NCORE + c
    zero16 = jnp.zeros((16,), jnp.float32)
    ninf16 = jnp.full((16,), -3.0e38, jnp.float32)

    @pl.loop(0, _FLAT // 16)
    def _(i):
        sum_v[pl.ds(16 * i, 16)] = zero16
        max_v[pl.ds(16 * i, 16)] = ninf16
        cnt_v[pl.ds(16 * i, 16)] = zero16

    @pl.when(wid < _N // _PROWS)
    def _():
        base = _PROWS * wid
        pltpu.sync_copy(h_hbm.at[pl.ds(base, _PROWS)], rows_v)
        pltpu.sync_copy(b_hbm.at[pl.ds(base, _PROWS)], bid_v)
        lane = lax.iota(jnp.int32, 16)
        ones16 = jnp.ones((16,), jnp.float32)

        def _rbody(r, carry):
            b = plsc.load_gather(bid_v, [jnp.full((16,), r, jnp.int32)])
            b64 = b * _H
            for j in range(_H // 16):
                idx = b64 + (16 * j) + lane
                chunk = rows_v[r, pl.ds(16 * j, 16)]
                plsc.addupdate_scatter(sum_v, [idx], chunk)
                plsc.addupdate_scatter(cnt_v, [idx], ones16)
                old = plsc.load_gather(max_v, [idx])
                plsc.store_scatter(max_v, [idx], jnp.maximum(old, chunk))
            return carry

        lax.fori_loop(0, _PROWS, _rbody, 0)

    pltpu.sync_copy(sum_v, osum.at[wid])
    pltpu.sync_copy(max_v, omax.at[wid])
    pltpu.sync_copy(cnt_v, ocnt.at[wid])


_PREC = lax.Precision.DEFAULT


def _proj_body(x_ref, w_ref, o_ref):
    o_ref[...] = jnp.dot(x_ref[...], w_ref[...],
                         preferred_element_type=jnp.float32, precision=_PREC)


def _mlp_body(has_proj, y_ref, p0_ref, p1_ref, eps_ref, ba_ref, wb_ref,
              bb_ref, g_ref, be_ref, rm_ref, rv_ref, *rest):
    if has_proj:
        wn_ref, o_ref = rest
    else:
        (o_ref,) = rest
    z = ((1.0 + eps_ref[0, 0]) * y_ref[...] + p0_ref[...] + p1_ref[...]
         + ba_ref[...])
    a = jnp.maximum(z, 0.0)
    u = jnp.dot(a, wb_ref[...], preferred_element_type=jnp.float32,
                precision=_PREC) + bb_ref[...]
    v = jnp.maximum(u, 0.0)
    hh = ((v - rm_ref[...]) / jnp.sqrt(rv_ref[...] + 1e-5) * g_ref[...]
          + be_ref[...])
    if has_proj:
        o_ref[...] = jnp.dot(hh, wn_ref[...],
                             preferred_element_type=jnp.float32,
                             precision=_PREC)
    else:
        o_ref[...] = hh


def _final_body(s_ref, m_ref, c_ref, wf_ref, bf_ref, o_ref):
    ssum = s_ref[0]
    mmax = m_ref[0]
    csum = c_ref[0]
    for i in range(1, _NCORE * _NSUB):
        ssum = ssum + s_ref[i]
        mmax = jnp.maximum(mmax, m_ref[i])
        csum = csum + c_ref[i]
    ssum = ssum[:_G]
    mmax = mmax[:_G]
    csum = csum[:_G]
    mean = ssum / jnp.maximum(csum, 1.0)
    mx = jnp.where(csum > 0.0, mmax, 0.0)
    pooled = jnp.concatenate([mean, mx], axis=1)
    logits = jnp.dot(pooled, wf_ref[...], preferred_element_type=jnp.float32,
                     precision=_PREC) + bf_ref[...]
    lmax = jnp.max(logits, axis=1, keepdims=True)
    shifted = logits - lmax
    lse = jnp.log(jnp.sum(jnp.exp(shifted), axis=1, keepdims=True))
    o_ref[...] = shifted - lse


def _proj(x, w):
    return pl.pallas_call(
        _proj_body,
        out_shape=jax.ShapeDtypeStruct((_N, _H), jnp.float32),
    )(x, w)


def _mlp(y, p0, p1, eps, ba, wb, bb, g, be, rm, rv, wn):
    args = [y, p0, p1, eps.reshape(1, 1), ba.reshape(1, _H), wb,
            bb.reshape(1, _H), g.reshape(1, _H), be.reshape(1, _H),
            rm.reshape(1, _H), rv.reshape(1, _H)]
    if wn is not None:
        args.append(wn)
    return pl.pallas_call(
        functools.partial(_mlp_body, wn is not None),
        out_shape=jax.ShapeDtypeStruct((_N, _H), jnp.float32),
    )(*args)


def _final(su, mx, ct, wf, bf):
    return pl.pallas_call(
        _final_body,
        out_shape=jax.ShapeDtypeStruct((_G, _C), jnp.float32),
    )(su, mx, ct, wf, bf.reshape(1, _C))


def kernel(x, edge_index, batch,
           W1a, b1a, W1b, b1b,
           W2a, b2a, W2b, b2b,
           W3a, b3a, W3b, b3b,
           eps1, eps2, eps3, eps4,
           g1, be1, g2, be2, g3, be3, g4, be4,
           rm1, rv1, rm2, rv2, rm3, rv3, rm4, rv4,
           Wf, bf):
    pad = jnp.concatenate(
        [jnp.zeros((1, _EPAD - _E), jnp.int32),
         jnp.full((1, _EPAD - _E), _N, jnp.int32)], axis=0)
    ei2 = jnp.concatenate([edge_index, pad], axis=1).reshape(
        2, _NCORE * _NSUB, _NK, _CHUNK)

    y1 = _proj(x, W1a)
    p = _agg_kernel(y1, ei2)
    y2 = _mlp(y1, p[0], p[1], eps1, b1a, W1b, b1b, g1, be1, rm1, rv1, W2a)
    p = _agg_kernel(y2, ei2)
    y3 = _mlp(y2, p[0], p[1], eps2, b2a, W2b, b2b, g2, be2, rm2, rv2, W3a)
    p = _agg_kernel(y3, ei2)
    y4 = _mlp(y3, p[0], p[1], eps3, b3a, W3b, b3b, g3, be3, rm3, rv3, W3a)
    p = _agg_kernel(y4, ei2)
    h4 = _mlp(y4, p[0], p[1], eps4, b3a, W3b, b3b, g4, be4, rm4, rv4, None)

    su, mx, ct = _pool_kernel(h4, batch)
    su = su.reshape(_NCORE * _NSUB, _GP, _H)
    mx = mx.reshape(_NCORE * _NSUB, _GP, _H)
    ct = ct.reshape(_NCORE * _NSUB, _GP, _H)
    return _final(su, mx, ct, Wf, bf)
